# Initial kernel scaffold; baseline (speedup 1.0000x reference)
import jax
import jax.numpy as jnp
from jax.experimental import pallas as pl


def kernel(latent, attr, w1, w2, w3, w4, w5, g1, b1, g2, b2, g3, b3, g4, b4):
    raise NotImplementedError("write your pallas kernel here")


# speedup vs baseline: 1.5292x; 1.5292x over previous
import functools

import jax
import jax.numpy as jnp
from jax.experimental import pallas as pl
from jax.experimental.pallas import tpu as pltpu


def _dense_bn_kernel(x_ref, w_ref, o_ref, st_ref, *, C1):
    y = jnp.dot(x_ref[...], w_ref[...], preferred_element_type=jnp.float32)
    o_ref[...] = y
    acc_s = None
    acc_q = None
    for j in range(16):
        blk = y[:, j * C1:(j + 1) * C1]
        acc_s = blk if acc_s is None else acc_s + blk
        acc_q = blk * blk if acc_q is None else acc_q + blk * blk
    st_ref[0, 0, :] = jnp.sum(acc_s, axis=0)
    st_ref[0, 1, :] = jnp.sum(acc_q, axis=0)


def _convt_kernel(x_ref, w_ref, ss_ref, *refs, H, W, Cout, BB, fuse_tanh):
    if fuse_tanh:
        o_ref, ppad_ref = refs
        st_ref = None
    else:
        o_ref, st_ref, ppad_ref = refs
    Cin = x_ref.shape[-1]
    N16 = 16 * Cout

    x = x_ref[...]
    xn = jnp.maximum(x * ss_ref[0, :] + ss_ref[1, :], 0.0)
    xn = xn.astype(w_ref.dtype)

    p = jnp.dot(xn.reshape(BB * H * W, Cin), w_ref[...],
                preferred_element_type=jnp.float32)

    ppad_ref[:, 0, :, :] = jnp.zeros((BB, W + 2, N16), jnp.float32)
    ppad_ref[:, H + 1, :, :] = jnp.zeros((BB, W + 2, N16), jnp.float32)
    ppad_ref[:, :, 0, :] = jnp.zeros((BB, H + 2, N16), jnp.float32)
    ppad_ref[:, :, W + 1, :] = jnp.zeros((BB, H + 2, N16), jnp.float32)
    ppad_ref[:, 1:H + 1, 1:W + 1, :] = p.reshape(BB, H, W, N16)

    ysum = None
    ysq = None
    for rh in range(2):
        halves = []
        for rw in range(2):
            acc = None
            for dh in range(2):
                for dw in range(2):
                    t = (3 - 2 * dh - rh) * 4 + (3 - 2 * dw - rw)
                    v = ppad_ref[:, rh + dh:rh + dh + H,
                                 rw + dw:rw + dw + W,
                                 t * Cout:(t + 1) * Cout]
                    acc = v if acc is None else acc + v
            halves.append(acc)
            if st_ref is not None:
                s = jnp.sum(acc, axis=(0, 1, 2))
                q = jnp.sum(acc * acc, axis=(0, 1, 2))
                ysum = s if ysum is None else ysum + s
                ysq = q if ysq is None else ysq + q
        pair = jnp.concatenate(halves, axis=-1)
        if fuse_tanh:
            pair = jnp.tanh(pair)
        o_ref[:, rh, :, :] = pair.reshape(BB * H, W, 2 * Cout)
    if st_ref is not None:
        st_ref[0, 0, :] = ysum
        st_ref[0, 1, :] = ysq


def _dense_bn_layer(x, wmat, C1):
    B, K = x.shape
    N = wmat.shape[1]
    BB = B // 2
    out, st = pl.pallas_call(
        functools.partial(_dense_bn_kernel, C1=C1),
        out_shape=(jax.ShapeDtypeStruct((B, N), jnp.float32),
                   jax.ShapeDtypeStruct((2, 2, C1), jnp.float32)),
        grid=(2,),
        in_specs=[pl.BlockSpec((BB, K), lambda b: (b, 0)),
                  pl.BlockSpec((K, N), lambda b: (0, 0))],
        out_specs=(pl.BlockSpec((BB, N), lambda b: (b, 0)),
                   pl.BlockSpec((1, 2, C1), lambda b: (b, 0, 0))),
        compiler_params=pltpu.CompilerParams(dimension_semantics=("parallel",)),
    )(x, wmat)
    return out, st


def _pick_bb(B, H, W, Cin, N16, cap_bytes=24 * 2 ** 20):
    cout2 = max(2 * (N16 // 16), 128)
    bb = B // 2
    while bb > 1:
        out_b = bb * H * 2 * W * cout2 * 4 * 2
        in_b = bb * H * W * max(Cin, 128) * 4 * 2
        scr_b = bb * (H + 2) * (W + 2) * max(N16, 128) * 4
        p_b = bb * H * W * max(N16, 128) * 4 * 2
        if out_b + in_b + scr_b + p_b <= cap_bytes:
            break
        bb //= 2
    return bb


def _convt_bn_layer(x_nhwc, wmat, scale_shift, *, fuse_tanh=False):
    B, H, W, Cin = x_nhwc.shape
    N16 = wmat.shape[1]
    Cout = N16 // 16
    BB = _pick_bb(B, H, W, Cin, N16)
    G = B // BB

    out_shape = [jax.ShapeDtypeStruct((B * H, 2, W, 2 * Cout), jnp.float32)]
    out_specs = [pl.BlockSpec((BB * H, 2, W, 2 * Cout), lambda b: (b, 0, 0, 0))]
    if not fuse_tanh:
        out_shape.append(jax.ShapeDtypeStruct((G, 2, Cout), jnp.float32))
        out_specs.append(pl.BlockSpec((1, 2, Cout), lambda b: (b, 0, 0)))

    res = pl.pallas_call(
        functools.partial(_convt_kernel, H=H, W=W, Cout=Cout, BB=BB,
                          fuse_tanh=fuse_tanh),
        out_shape=tuple(out_shape),
        grid=(G,),
        in_specs=[
            pl.BlockSpec((BB, H, W, Cin), lambda b: (b, 0, 0, 0)),
            pl.BlockSpec((Cin, N16), lambda b: (0, 0)),
            pl.BlockSpec((2, Cin), lambda b: (0, 0)),
        ],
        out_specs=tuple(out_specs),
        scratch_shapes=[pltpu.VMEM((BB, H + 2, W + 2, N16), jnp.float32)],
        compiler_params=pltpu.CompilerParams(dimension_semantics=("parallel",)),
    )(x_nhwc, wmat, scale_shift)

    y = res[0].reshape(B, 2 * H, 2 * W, Cout)
    return (y, None) if fuse_tanh else (y, res[1])


def _w_to_mat(w):
    cin, cout = w.shape[0], w.shape[1]
    return jnp.transpose(w, (0, 2, 3, 1)).reshape(cin, 16 * cout)


def _stats_to_ss(st, count, gamma, beta, eps=1e-5):
    s = jnp.sum(st[:, 0, :], axis=0)
    q = jnp.sum(st[:, 1, :], axis=0)
    mean = s / count
    var = q / count - mean * mean
    inv = jax.lax.rsqrt(var + eps)
    scale = gamma * inv
    return jnp.stack([scale, beta - mean * scale], axis=0).astype(jnp.float32)


def kernel(latent, attr, w1, w2, w3, w4, w5, g1, b1, g2, b2, g3, b3, g4, b4):
    B = latent.shape[0]
    x = jnp.concatenate([latent, attr], axis=1).astype(jnp.float32)

    wm = {i: _w_to_mat(w).astype(jnp.bfloat16)
          for i, w in ((1, w1), (2, w2), (3, w3), (4, w4), (5, w5))}

    C1 = w1.shape[1]
    y2d, st = _dense_bn_layer(x.astype(jnp.bfloat16), wm[1], C1)
    y = y2d.reshape(B, 4, 4, C1)
    ss = _stats_to_ss(st, B * 16, g1, b1)

    for wmat, g, b in ((wm[2], g2, b2), (wm[3], g3, b3), (wm[4], g4, b4)):
        y, st = _convt_bn_layer(y, wmat, ss, fuse_tanh=False)
        ss = _stats_to_ss(st, y.shape[0] * y.shape[1] * y.shape[2], g, b)

    y, _ = _convt_bn_layer(y, wm[5], ss, fuse_tanh=True)
    return jnp.transpose(y, (0, 3, 1, 2))


# speedup vs baseline: 3.8572x; 2.5223x over previous
import functools

import jax
import jax.numpy as jnp
import numpy as np
from jax.experimental import pallas as pl
from jax.experimental.pallas import tpu as pltpu


def _dense_bn_kernel(x_ref, w_ref, o_ref, st_ref, *, C1):
    y = jnp.dot(x_ref[...], w_ref[...], preferred_element_type=jnp.float32)
    o_ref[...] = y
    acc_s = None
    acc_q = None
    for j in range(16):
        blk = y[:, j * C1:(j + 1) * C1]
        acc_s = blk if acc_s is None else acc_s + blk
        acc_q = blk * blk if acc_q is None else acc_q + blk * blk
    st_ref[0, 0, :] = jnp.sum(acc_s, axis=0)
    st_ref[0, 1, :] = jnp.sum(acc_q, axis=0)


def _convt_kernel(x_ref, w_ref, ss_ref, o_ref, st_ref, xsh_ref, *, H, W,
                  Cout, BB):
    Cin = x_ref.shape[-1]
    M = BB * H * W

    x = x_ref[...]
    xn = jnp.maximum(x * ss_ref[0, :] + ss_ref[1, :], 0.0)

    zcol = jnp.zeros((BB, H, 1, Cin), jnp.float32)
    xl = jnp.concatenate([xn[:, :, 1:, :], zcol], axis=2)
    xr = jnp.concatenate([zcol, xn[:, :, :W - 1, :]], axis=2)
    zrow = jnp.zeros((BB, W, 3 * Cin), jnp.bfloat16)
    xsh_ref[:, 0, :, :] = zrow
    xsh_ref[:, H + 1, :, :] = zrow
    xsh_ref[:, 1:H + 1, :, 0:Cin] = xr.astype(jnp.bfloat16)
    xsh_ref[:, 1:H + 1, :, Cin:2 * Cin] = xn.astype(jnp.bfloat16)
    xsh_ref[:, 1:H + 1, :, 2 * Cin:] = xl.astype(jnp.bfloat16)

    ysum = None
    ysq = None
    for rh in range(2):
        for rw in range(2):
            p = 2 * rh + rw
            lo = xsh_ref[:, rh:rh + H, :, rw * Cin:(rw + 2) * Cin]
            hi = xsh_ref[:, rh + 1:rh + 1 + H, :, rw * Cin:(rw + 2) * Cin]
            acc = jnp.dot(lo.reshape(M, 2 * Cin), w_ref[p, :2 * Cin, :],
                          preferred_element_type=jnp.float32)
            acc = acc + jnp.dot(hi.reshape(M, 2 * Cin), w_ref[p, 2 * Cin:, :],
                                preferred_element_type=jnp.float32)
            s = jnp.sum(acc, axis=0)
            q = jnp.sum(acc * acc, axis=0)
            ysum = s if ysum is None else ysum + s
            ysq = q if ysq is None else ysq + q
            o_ref[:, rh, :, rw * Cout:(rw + 1) * Cout] = (
                acc.reshape(BB * H, W, Cout))
    st_ref[0, 0, :] = ysum
    st_ref[0, 1, :] = ysq


def _rgb_kernel(x_ref, wt_ref, ss_ref, e_ref, o_ref, *, H, W, BB):
    Cin = x_ref.shape[-1]
    M = BB * H * W

    x = x_ref[...]
    xn = jnp.maximum(x * ss_ref[0, :] + ss_ref[1, :], 0.0)
    xn2 = xn.astype(wt_ref.dtype).reshape(M, Cin)

    pt = jax.lax.dot_general(wt_ref[...], xn2, (((1,), (1,)), ((), ())),
                             preferred_element_type=jnp.float32)

    lane = jax.lax.broadcasted_iota(jnp.int32, (1, M), 1)
    j_idx = lane & (W - 1)
    a_idx = (lane // W) & (H - 1)

    rows = []
    for rh in range(2):
        for rw in range(2):
            acc = None
            for dh in range(2):
                for dw in range(2):
                    t = (3 - 2 * dh - rh) * 4 + (3 - 2 * dw - rw)
                    sh = dh + rh - 1
                    sw = dw + rw - 1
                    v = pt[3 * t:3 * t + 3, :]
                    shift = sh * W + sw
                    if shift:
                        v = pltpu.roll(v, (-shift) % M, axis=1)
                    ok = None
                    if sh:
                        ok = (a_idx + sh >= 0) & (a_idx + sh < H)
                    if sw:
                        okw = (j_idx + sw >= 0) & (j_idx + sw < W)
                        ok = okw if ok is None else ok & okw
                    if ok is not None:
                        v = jnp.where(ok, v, 0.0)
                    acc = v if acc is None else acc + v
            rows.append(jnp.tanh(acc))

    cat = jnp.concatenate(
        [r.reshape(3 * (M // 128), 128) for r in rows], axis=-1)
    out = jnp.dot(cat, e_ref[...], preferred_element_type=jnp.float32)
    o_ref[...] = out.reshape(3, M // 128, 512)


def _dense_bn_layer(x, wmat, C1):
    B, K = x.shape
    N = wmat.shape[1]
    BB = B // 2
    out, st = pl.pallas_call(
        functools.partial(_dense_bn_kernel, C1=C1),
        out_shape=(jax.ShapeDtypeStruct((B, N), jnp.float32),
                   jax.ShapeDtypeStruct((2, 2, C1), jnp.float32)),
        grid=(2,),
        in_specs=[pl.BlockSpec((BB, K), lambda b: (b, 0)),
                  pl.BlockSpec((K, N), lambda b: (0, 0))],
        out_specs=(pl.BlockSpec((BB, N), lambda b: (b, 0)),
                   pl.BlockSpec((1, 2, C1), lambda b: (b, 0, 0))),
        compiler_params=pltpu.CompilerParams(dimension_semantics=("parallel",)),
    )(x, wmat)
    return out, st


def _pick_bb(B, H, W, Cin, Cout, cap_bytes=20 * 2 ** 20):
    bb = B // 2
    while bb > 1:
        m = bb * H * W
        out_b = bb * H * 2 * W * max(2 * Cout, 128) * 4 * 2
        in_b = m * max(Cin, 128) * 4 * 2
        scr_b = bb * (H + 2) * W * 3 * max(Cin, 128) * 2
        acc_b = m * max(Cout, 128) * 4 * 2
        if out_b + in_b + scr_b + acc_b <= cap_bytes:
            break
        bb //= 2
    return bb


def _convt_bn_layer(x_nhwc, wph, scale_shift):
    B, H, W, Cin = x_nhwc.shape
    Cout = wph.shape[2]
    BB = _pick_bb(B, H, W, Cin, Cout)
    G = B // BB

    res = pl.pallas_call(
        functools.partial(_convt_kernel, H=H, W=W, Cout=Cout, BB=BB),
        out_shape=(jax.ShapeDtypeStruct((B * H, 2, W, 2 * Cout), jnp.float32),
                   jax.ShapeDtypeStruct((G, 2, Cout), jnp.float32)),
        grid=(G,),
        in_specs=[
            pl.BlockSpec((BB, H, W, Cin), lambda b: (b, 0, 0, 0)),
            pl.BlockSpec((4, 4 * Cin, Cout), lambda b: (0, 0, 0)),
            pl.BlockSpec((2, Cin), lambda b: (0, 0)),
        ],
        out_specs=(pl.BlockSpec((BB * H, 2, W, 2 * Cout),
                                lambda b: (b, 0, 0, 0)),
                   pl.BlockSpec((1, 2, Cout), lambda b: (b, 0, 0))),
        scratch_shapes=[pltpu.VMEM((BB, H + 2, W, 3 * Cin), jnp.bfloat16)],
        compiler_params=pltpu.CompilerParams(dimension_semantics=("parallel",)),
    )(x_nhwc, wph, scale_shift)

    return res[0].reshape(B, 2 * H, 2 * W, Cout), res[1]


def _phase_interleave_matrix():
    e = np.zeros((512, 512), np.float32)
    for q_out in range(512):
        om_loc, rem = divmod(q_out, 256)
        half, rem2 = divmod(rem, 128)
        rh, u = divmod(rem2, 64)
        j, rw = divmod(u, 2)
        m_local = 32 * (2 * om_loc + half) + j
        e[128 * (2 * rh + rw) + m_local, q_out] = 1.0
    return jnp.asarray(e)


def _rgb_layer(x_nhwc, wt, scale_shift, emat):
    B, H, W, Cin = x_nhwc.shape
    BB = 8 if B % 8 == 0 else (4 if B % 4 == 0 else B // 2)
    G = B // BB
    M = BB * H * W

    out = pl.pallas_call(
        functools.partial(_rgb_kernel, H=H, W=W, BB=BB),
        out_shape=jax.ShapeDtypeStruct((3, B * (H * W) // 128, 512),
                                       jnp.float32),
        grid=(G,),
        in_specs=[
            pl.BlockSpec((BB, H, W, Cin), lambda b: (b, 0, 0, 0)),
            pl.BlockSpec((48, Cin), lambda b: (0, 0)),
            pl.BlockSpec((2, Cin), lambda b: (0, 0)),
            pl.BlockSpec((512, 512), lambda b: (0, 0)),
        ],
        out_specs=pl.BlockSpec((3, M // 128, 512), lambda b: (0, b, 0)),
        compiler_params=pltpu.CompilerParams(dimension_semantics=("parallel",)),
    )(x_nhwc, wt, scale_shift, emat)

    return out.reshape(3, B, 2 * H, 2 * W)


def _w_to_mat(w):
    cin, cout = w.shape[0], w.shape[1]
    wb = w.astype(jnp.bfloat16)
    return jnp.transpose(wb, (0, 2, 3, 1)).reshape(cin, 16 * cout)


def _w_to_phase_mats(w):
    wf = jnp.transpose(w.astype(jnp.bfloat16), (2, 3, 0, 1))
    phases = []
    for rh in range(2):
        for rw in range(2):
            taps = [wf[1 - 2 * sh + rh, 1 - 2 * sw + rw]
                    for sh in (rh - 1, rh) for sw in (rw - 1, rw)]
            phases.append(jnp.concatenate(taps, axis=0))
    return jnp.stack(phases, axis=0)


def _stats_to_ss(st, count, gamma, beta, eps=1e-5):
    s = jnp.sum(st[:, 0, :], axis=0)
    q = jnp.sum(st[:, 1, :], axis=0)
    mean = s / count
    var = q / count - mean * mean
    inv = jax.lax.rsqrt(var + eps)
    scale = gamma * inv
    return jnp.stack([scale, beta - mean * scale], axis=0).astype(jnp.float32)


def kernel(latent, attr, w1, w2, w3, w4, w5, g1, b1, g2, b2, g3, b3, g4, b4):
    B = latent.shape[0]
    x = jnp.concatenate([latent, attr], axis=1).astype(jnp.float32)

    wm1 = _w_to_mat(w1)
    wph = {i: _w_to_phase_mats(w) for i, w in ((2, w2), (3, w3), (4, w4))}
    wt5 = jnp.transpose(w5.astype(jnp.bfloat16),
                        (2, 3, 1, 0)).reshape(48, w5.shape[0])
    emat = _phase_interleave_matrix()

    C1 = w1.shape[1]
    y2d, st = _dense_bn_layer(x.astype(jnp.bfloat16), wm1, C1)
    y = y2d.reshape(B, 4, 4, C1)
    ss = _stats_to_ss(st, B * 16, g1, b1)

    for wmat, g, b in ((wph[2], g2, b2), (wph[3], g3, b3), (wph[4], g4, b4)):
        y, st = _convt_bn_layer(y, wmat, ss)
        ss = _stats_to_ss(st, y.shape[0] * y.shape[1] * y.shape[2], g, b)

    y5 = _rgb_layer(y, wt5, ss, emat)
    return jnp.transpose(y5, (1, 0, 2, 3))


# speedup vs baseline: 4.2305x; 1.0968x over previous
import functools

import jax
import jax.numpy as jnp
import numpy as np
from jax.experimental import pallas as pl
from jax.experimental.pallas import tpu as pltpu


def _dense_bn_kernel(x_ref, w_ref, o_ref, st_ref, *, C1):
    y = jnp.dot(x_ref[...], w_ref[...], preferred_element_type=jnp.float32)
    o_ref[...] = y
    acc_s = None
    acc_q = None
    for j in range(16):
        blk = y[:, j * C1:(j + 1) * C1]
        acc_s = blk if acc_s is None else acc_s + blk
        acc_q = blk * blk if acc_q is None else acc_q + blk * blk
    st_ref[0, 0, :] = jnp.sum(acc_s, axis=0)
    st_ref[0, 1, :] = jnp.sum(acc_q, axis=0)


def _convt_kernel(x_ref, w_ref, ss_ref, o_ref, st_ref, xsh_ref, *, H, W,
                  Cin, Cout, BB):
    M = BB * H * W

    x = x_ref[...]
    xn = jnp.maximum(x * ss_ref[0, :] + ss_ref[1, :], 0.0)
    xn = xn.reshape(BB, H, W, Cin)

    zcol = jnp.zeros((BB, H, 1, Cin), jnp.float32)
    xl = jnp.concatenate([xn[:, :, 1:, :], zcol], axis=2)
    xr = jnp.concatenate([zcol, xn[:, :, :W - 1, :]], axis=2)
    zrow = jnp.zeros((BB, W, 3 * Cin), jnp.bfloat16)
    xsh_ref[:, 0, :, :] = zrow
    xsh_ref[:, H + 1, :, :] = zrow
    xsh_ref[:, 1:H + 1, :, 0:Cin] = xr.astype(jnp.bfloat16)
    xsh_ref[:, 1:H + 1, :, Cin:2 * Cin] = xn.astype(jnp.bfloat16)
    xsh_ref[:, 1:H + 1, :, 2 * Cin:] = xl.astype(jnp.bfloat16)

    ysum = None
    ysq = None
    for rh in range(2):
        for rw in range(2):
            p = 2 * rh + rw
            lo = xsh_ref[:, rh:rh + H, :, rw * Cin:(rw + 2) * Cin]
            hi = xsh_ref[:, rh + 1:rh + 1 + H, :, rw * Cin:(rw + 2) * Cin]
            acc = jnp.dot(lo.reshape(M, 2 * Cin), w_ref[p, :2 * Cin, :],
                          preferred_element_type=jnp.float32)
            acc = acc + jnp.dot(hi.reshape(M, 2 * Cin), w_ref[p, 2 * Cin:, :],
                                preferred_element_type=jnp.float32)
            s = jnp.sum(acc, axis=0)
            q = jnp.sum(acc * acc, axis=0)
            ysum = s if ysum is None else ysum + s
            ysq = q if ysq is None else ysq + q
            o_ref[:, rh, :, rw * Cout:(rw + 1) * Cout] = (
                acc.reshape(BB * H, W, Cout))
    st_ref[0, 0, :] = ysum
    st_ref[0, 1, :] = ysq


def _rgb_kernel(x_ref, wt_ref, ss_ref, e_ref, o_ref, *, H, W, BB):
    Cin = wt_ref.shape[-1]
    M = BB * H * W

    x = x_ref[...]
    xn = jnp.maximum(x * ss_ref[0, :] + ss_ref[1, :], 0.0)
    xn2 = xn.reshape(M, Cin).astype(wt_ref.dtype)

    pt = jax.lax.dot_general(wt_ref[...], xn2, (((1,), (1,)), ((), ())),
                             preferred_element_type=jnp.float32)

    lane = jax.lax.broadcasted_iota(jnp.int32, (1, M), 1)
    j_idx = lane & (W - 1)
    a_idx = (lane // W) & (H - 1)

    rows = []
    for rh in range(2):
        for rw in range(2):
            acc = None
            for dh in range(2):
                for dw in range(2):
                    t = (3 - 2 * dh - rh) * 4 + (3 - 2 * dw - rw)
                    sh = dh + rh - 1
                    sw = dw + rw - 1
                    v = pt[3 * t:3 * t + 3, :]
                    shift = sh * W + sw
                    if shift:
                        v = pltpu.roll(v, (-shift) % M, axis=1)
                    ok = None
                    if sh:
                        ok = (a_idx + sh >= 0) & (a_idx + sh < H)
                    if sw:
                        okw = (j_idx + sw >= 0) & (j_idx + sw < W)
                        ok = okw if ok is None else ok & okw
                    if ok is not None:
                        v = jnp.where(ok, v, 0.0)
                    acc = v if acc is None else acc + v
            rows.append(jnp.tanh(acc))

    cat = jnp.concatenate(
        [r.reshape(3 * (M // 128), 128) for r in rows], axis=-1)
    out = jnp.dot(cat, e_ref[...], preferred_element_type=jnp.float32)
    o_ref[...] = out.reshape(3, M // 128, 512)


def _dense_bn_layer(x, wmat, C1):
    B, K = x.shape
    N = wmat.shape[1]
    BB = B // 2
    out, st = pl.pallas_call(
        functools.partial(_dense_bn_kernel, C1=C1),
        out_shape=(jax.ShapeDtypeStruct((B, N), jnp.float32),
                   jax.ShapeDtypeStruct((2, 2, C1), jnp.float32)),
        grid=(2,),
        in_specs=[pl.BlockSpec((BB, K), lambda b: (b, 0)),
                  pl.BlockSpec((K, N), lambda b: (0, 0))],
        out_specs=(pl.BlockSpec((BB, N), lambda b: (b, 0)),
                   pl.BlockSpec((1, 2, C1), lambda b: (b, 0, 0))),
        compiler_params=pltpu.CompilerParams(dimension_semantics=("parallel",)),
    )(x, wmat)
    return out, st


def _pick_bb(B, H, W, Cin, Cout, cap_bytes=20 * 2 ** 20):
    bb = B // 2
    while bb > 1:
        m = bb * H * W
        out_b = bb * H * 2 * W * max(2 * Cout, 128) * 4 * 2
        in_b = m * max(Cin, 128) * 4 * 2
        scr_b = bb * (H + 2) * W * 3 * max(Cin, 128) * 2
        acc_b = m * max(Cout, 128) * 4 * 2
        if out_b + in_b + scr_b + acc_b <= cap_bytes:
            break
        bb //= 2
    return bb


def _convt_bn_layer(x_raw, wph, scale_shift, B, H, W, Cin):
    Cout = wph.shape[2]
    BB = _pick_bb(B, H, W, Cin, Cout)
    G = B // BB

    xdims = x_raw.shape
    xblock = (xdims[0] // G,) + xdims[1:]
    nlead = len(xdims) - 1
    ss_t = jnp.tile(scale_shift, (1, xdims[-1] // Cin))

    res = pl.pallas_call(
        functools.partial(_convt_kernel, H=H, W=W, Cin=Cin, Cout=Cout, BB=BB),
        out_shape=(jax.ShapeDtypeStruct((B * H, 2, W, 2 * Cout), jnp.float32),
                   jax.ShapeDtypeStruct((G, 2, Cout), jnp.float32)),
        grid=(G,),
        in_specs=[
            pl.BlockSpec(xblock, lambda b: (b,) + (0,) * nlead),
            pl.BlockSpec((4, 4 * Cin, Cout), lambda b: (0, 0, 0)),
            pl.BlockSpec((2, xdims[-1]), lambda b: (0, 0)),
        ],
        out_specs=(pl.BlockSpec((BB * H, 2, W, 2 * Cout),
                                lambda b: (b, 0, 0, 0)),
                   pl.BlockSpec((1, 2, Cout), lambda b: (b, 0, 0))),
        scratch_shapes=[pltpu.VMEM((BB, H + 2, W, 3 * Cin), jnp.bfloat16)],
        compiler_params=pltpu.CompilerParams(dimension_semantics=("parallel",)),
    )(x_raw, wph, ss_t)

    return res[0], res[1]


def _phase_interleave_matrix():
    e = np.zeros((512, 512), np.float32)
    for q_out in range(512):
        om_loc, rem = divmod(q_out, 256)
        half, rem2 = divmod(rem, 128)
        rh, u = divmod(rem2, 64)
        j, rw = divmod(u, 2)
        m_local = 32 * (2 * om_loc + half) + j
        e[128 * (2 * rh + rw) + m_local, q_out] = 1.0
    return jnp.asarray(e)


def _rgb_layer(x_raw, wt, scale_shift, emat, B, H, W, Cin):
    BB = 8 if B % 8 == 0 else (4 if B % 4 == 0 else B // 2)
    G = B // BB
    M = BB * H * W

    xdims = x_raw.shape
    xblock = (xdims[0] // G,) + xdims[1:]
    nlead = len(xdims) - 1
    ss_t = jnp.tile(scale_shift, (1, xdims[-1] // Cin))

    out = pl.pallas_call(
        functools.partial(_rgb_kernel, H=H, W=W, BB=BB),
        out_shape=jax.ShapeDtypeStruct((3, B * (H * W) // 128, 512),
                                       jnp.float32),
        grid=(G,),
        in_specs=[
            pl.BlockSpec(xblock, lambda b: (b,) + (0,) * nlead),
            pl.BlockSpec((48, Cin), lambda b: (0, 0)),
            pl.BlockSpec((2, xdims[-1]), lambda b: (0, 0)),
            pl.BlockSpec((512, 512), lambda b: (0, 0)),
        ],
        out_specs=pl.BlockSpec((3, M // 128, 512), lambda b: (0, b, 0)),
        compiler_params=pltpu.CompilerParams(dimension_semantics=("parallel",)),
    )(x_raw, wt, ss_t, emat)

    return out.reshape(3, B, 2 * H, 2 * W)


def _w_to_mat(w):
    cin, cout = w.shape[0], w.shape[1]
    wb = w.astype(jnp.bfloat16)
    return jnp.transpose(wb, (0, 2, 3, 1)).reshape(cin, 16 * cout)


def _w_to_phase_mats(w):
    wb = w.astype(jnp.bfloat16)
    phases = []
    for rh in range(2):
        for rw in range(2):
            taps = [wb[:, :, 1 - 2 * sh + rh, 1 - 2 * sw + rw]
                    for sh in (rh - 1, rh) for sw in (rw - 1, rw)]
            phases.append(jnp.concatenate(taps, axis=0))
    return jnp.stack(phases, axis=0)


def _stats_to_ss(st, count, gamma, beta, eps=1e-5):
    s = jnp.sum(st[:, 0, :], axis=0)
    q = jnp.sum(st[:, 1, :], axis=0)
    mean = s / count
    var = q / count - mean * mean
    inv = jax.lax.rsqrt(var + eps)
    scale = gamma * inv
    return jnp.stack([scale, beta - mean * scale], axis=0).astype(jnp.float32)


def kernel(latent, attr, w1, w2, w3, w4, w5, g1, b1, g2, b2, g3, b3, g4, b4):
    B = latent.shape[0]
    x = jnp.concatenate([latent, attr], axis=1).astype(jnp.float32)

    wm1 = _w_to_mat(w1)
    wph = {i: _w_to_phase_mats(w) for i, w in ((2, w2), (3, w3), (4, w4))}
    wt5 = jnp.transpose(w5.astype(jnp.bfloat16),
                        (2, 3, 1, 0)).reshape(48, w5.shape[0])
    emat = _phase_interleave_matrix()

    C1 = w1.shape[1]
    y, st = _dense_bn_layer(x.astype(jnp.bfloat16), wm1, C1)
    ss = _stats_to_ss(st, B * 16, g1, b1)

    H, W, Cin = 4, 4, C1
    for wmat, g, b in ((wph[2], g2, b2), (wph[3], g3, b3), (wph[4], g4, b4)):
        y, st = _convt_bn_layer(y, wmat, ss, B, H, W, Cin)
        H, W, Cin = 2 * H, 2 * W, Cin // 2
        ss = _stats_to_ss(st, B * H * W, g, b)

    y5 = _rgb_layer(y.reshape(B, H, W, Cin), wt5, ss, emat, B, H, W, Cin)
    return jnp.transpose(y5, (1, 0, 2, 3))


# speedup vs baseline: 5.0233x; 1.1874x over previous
import functools

import jax
import jax.numpy as jnp
import numpy as np
from jax.experimental import pallas as pl
from jax.experimental.pallas import tpu as pltpu


def _dense_bn_kernel(x_ref, w_ref, o_ref, st_ref, *, C1):
    y = jnp.dot(x_ref[...], w_ref[...], preferred_element_type=jnp.float32)
    o_ref[...] = y
    acc_s = None
    acc_q = None
    for j in range(16):
        blk = y[:, j * C1:(j + 1) * C1]
        acc_s = blk if acc_s is None else acc_s + blk
        acc_q = blk * blk if acc_q is None else acc_q + blk * blk
    st_ref[0, 0, :] = jnp.sum(acc_s, axis=0)
    st_ref[0, 1, :] = jnp.sum(acc_q, axis=0)


def _convt_kernel(x_ref, w_ref, ss_ref, o_ref, st_ref, xsh_ref, *, H, W,
                  Cin, Cout, BB):
    M = BB * H * W

    x = x_ref[...]
    xn = jnp.maximum(x * ss_ref[0, :] + ss_ref[1, :], 0.0)
    xn = xn.reshape(BB, H, W, Cin)

    zcol = jnp.zeros((BB, H, 1, Cin), jnp.float32)
    xl = jnp.concatenate([xn[:, :, 1:, :], zcol], axis=2)
    xr = jnp.concatenate([zcol, xn[:, :, :W - 1, :]], axis=2)
    zrow = jnp.zeros((BB, W, 3 * Cin), jnp.bfloat16)
    xsh_ref[:, 0, :, :] = zrow
    xsh_ref[:, H + 1, :, :] = zrow
    xsh_ref[:, 1:H + 1, :, 0:Cin] = xr.astype(jnp.bfloat16)
    xsh_ref[:, 1:H + 1, :, Cin:2 * Cin] = xn.astype(jnp.bfloat16)
    xsh_ref[:, 1:H + 1, :, 2 * Cin:] = xl.astype(jnp.bfloat16)

    ysum = None
    ysq = None
    for rh in range(2):
        for rw in range(2):
            p = 2 * rh + rw
            lo = xsh_ref[:, rh:rh + H, :, rw * Cin:(rw + 2) * Cin]
            hi = xsh_ref[:, rh + 1:rh + 1 + H, :, rw * Cin:(rw + 2) * Cin]
            acc = jnp.dot(lo.reshape(M, 2 * Cin), w_ref[p, :2 * Cin, :],
                          preferred_element_type=jnp.float32)
            acc = acc + jnp.dot(hi.reshape(M, 2 * Cin), w_ref[p, 2 * Cin:, :],
                                preferred_element_type=jnp.float32)
            s = jnp.sum(acc, axis=0)
            q = jnp.sum(acc * acc, axis=0)
            ysum = s if ysum is None else ysum + s
            ysq = q if ysq is None else ysq + q
            o_ref[:, rh, :, rw * Cout:(rw + 1) * Cout] = (
                acc.reshape(BB * H, W, Cout))
    st_ref[0, 0, :] = ysum
    st_ref[0, 1, :] = ysq


def _rgb_kernel(x_ref, wt_ref, ss_ref, e_ref, o_ref, *, H, W, BB):
    M = BB * H * W
    MP = M // 2

    x = x_ref[...]
    xn = jnp.maximum(x * ss_ref[0, :] + ss_ref[1, :], 0.0)
    xnp = xn.astype(wt_ref.dtype).reshape(MP, wt_ref.shape[1])

    pt = jax.lax.dot_general(wt_ref[...], xnp, (((1,), (1,)), ((), ())),
                             preferred_element_type=jnp.float32)

    lane = jax.lax.broadcasted_iota(jnp.int32, (1, MP), 1)
    j2 = lane & (W // 2 - 1)
    a_idx = (lane // (W // 2)) & (H - 1)

    rows = []
    for rh in range(2):
        for rw in range(2):
            accs = [None, None]
            for dh in range(2):
                for dw in range(2):
                    t = (3 - 2 * dh - rh) * 4 + (3 - 2 * dw - rw)
                    sh = dh + rh - 1
                    sw = dw + rw - 1
                    s = sh * W + sw
                    for q in range(2):
                        qs = (q + s) & 1
                        d = (q + s - qs) // 2
                        v = pt[48 * qs + 3 * t:48 * qs + 3 * t + 3, :]
                        if d:
                            v = pltpu.roll(v, (-d) % MP, axis=1)
                        ok = None
                        if sh:
                            ok = (a_idx + sh >= 0) & (a_idx + sh < H)
                        if sw:
                            jq = 2 * j2 + q
                            okw = (jq + sw >= 0) & (jq + sw < W)
                            ok = okw if ok is None else ok & okw
                        if ok is not None:
                            v = jnp.where(ok, v, 0.0)
                        accs[q] = v if accs[q] is None else accs[q] + v
            rows.append(jnp.tanh(accs[0]))
            rows.append(jnp.tanh(accs[1]))

    cat = jnp.concatenate(
        [r.astype(jnp.bfloat16).reshape(3 * MP // 128, 128) for r in rows],
        axis=-1)
    out = jnp.dot(cat, e_ref[...], preferred_element_type=jnp.float32)
    o_ref[...] = out.reshape(3, MP // 128, 1024)


def _dense_bn_layer(x, wmat, C1):
    B, K = x.shape
    N = wmat.shape[1]
    BB = B // 2
    out, st = pl.pallas_call(
        functools.partial(_dense_bn_kernel, C1=C1),
        out_shape=(jax.ShapeDtypeStruct((B, N), jnp.float32),
                   jax.ShapeDtypeStruct((2, 2, C1), jnp.float32)),
        grid=(2,),
        in_specs=[pl.BlockSpec((BB, K), lambda b: (b, 0)),
                  pl.BlockSpec((K, N), lambda b: (0, 0))],
        out_specs=(pl.BlockSpec((BB, N), lambda b: (b, 0)),
                   pl.BlockSpec((1, 2, C1), lambda b: (b, 0, 0))),
        compiler_params=pltpu.CompilerParams(dimension_semantics=("parallel",)),
    )(x, wmat)
    return out, st


def _pick_bb(B, H, W, Cin, Cout, cap_bytes=20 * 2 ** 20):
    bb = B // 2
    while bb > 1:
        m = bb * H * W
        out_b = bb * H * 2 * W * max(2 * Cout, 128) * 4 * 2
        in_b = m * max(Cin, 128) * 4 * 2
        scr_b = bb * (H + 2) * W * 3 * max(Cin, 128) * 2
        acc_b = m * max(Cout, 128) * 4 * 2
        if out_b + in_b + scr_b + acc_b <= cap_bytes:
            break
        bb //= 2
    return bb


def _convt_bn_layer(x_raw, wph, scale_shift, B, H, W, Cin):
    Cout = wph.shape[2]
    BB = _pick_bb(B, H, W, Cin, Cout)
    G = B // BB

    xdims = x_raw.shape
    xblock = (xdims[0] // G,) + xdims[1:]
    nlead = len(xdims) - 1
    ss_t = jnp.tile(scale_shift, (1, xdims[-1] // Cin))

    res = pl.pallas_call(
        functools.partial(_convt_kernel, H=H, W=W, Cin=Cin, Cout=Cout, BB=BB),
        out_shape=(jax.ShapeDtypeStruct((B * H, 2, W, 2 * Cout), jnp.float32),
                   jax.ShapeDtypeStruct((G, 2, Cout), jnp.float32)),
        grid=(G,),
        in_specs=[
            pl.BlockSpec(xblock, lambda b: (b,) + (0,) * nlead),
            pl.BlockSpec((4, 4 * Cin, Cout), lambda b: (0, 0, 0)),
            pl.BlockSpec((2, xdims[-1]), lambda b: (0, 0)),
        ],
        out_specs=(pl.BlockSpec((BB * H, 2, W, 2 * Cout),
                                lambda b: (b, 0, 0, 0)),
                   pl.BlockSpec((1, 2, Cout), lambda b: (b, 0, 0))),
        scratch_shapes=[pltpu.VMEM((BB, H + 2, W, 3 * Cin), jnp.bfloat16)],
        compiler_params=pltpu.CompilerParams(dimension_semantics=("parallel",)),
    )(x_raw, wph, ss_t)

    return res[0], res[1]


def _phase_interleave_matrix():
    e = np.zeros((1024, 1024), np.float32)
    for q_out in range(1024):
        da, rem = divmod(q_out, 128)
        rh, u = divmod(rem, 64)
        j, rw = divmod(u, 2)
        p = 2 * rh + rw
        v = 32 * da + j
        e[128 * (2 * p + (v & 1)) + v // 2, q_out] = 1.0
    return jnp.asarray(e).astype(jnp.bfloat16)


def _rgb_layer(x_raw, wt, scale_shift, emat, B, H, W, Cin):
    BB = 8 if B % 8 == 0 else (4 if B % 4 == 0 else B // 2)
    G = B // BB
    M = BB * H * W

    xdims = x_raw.shape
    xblock = (xdims[0] // G,) + xdims[1:]
    nlead = len(xdims) - 1
    ss_t = jnp.tile(scale_shift, (1, xdims[-1] // Cin))

    out = pl.pallas_call(
        functools.partial(_rgb_kernel, H=H, W=W, BB=BB),
        out_shape=jax.ShapeDtypeStruct((3, B * (H * W) // 256, 1024),
                                       jnp.float32),
        grid=(G,),
        in_specs=[
            pl.BlockSpec(xblock, lambda b: (b,) + (0,) * nlead),
            pl.BlockSpec(wt.shape, lambda b: (0, 0)),
            pl.BlockSpec((2, xdims[-1]), lambda b: (0, 0)),
            pl.BlockSpec((1024, 1024), lambda b: (0, 0)),
        ],
        out_specs=pl.BlockSpec((3, M // 256, 1024), lambda b: (0, b, 0)),
        compiler_params=pltpu.CompilerParams(dimension_semantics=("parallel",)),
    )(x_raw, wt, ss_t, emat)

    return out.reshape(3, B, 2 * H, 2 * W)


def _w_to_mat(w):
    cin, cout = w.shape[0], w.shape[1]
    wb = w.astype(jnp.bfloat16)
    return jnp.transpose(wb, (0, 2, 3, 1)).reshape(cin, 16 * cout)


def _w_to_phase_mats(w):
    wb = w.astype(jnp.bfloat16)
    phases = []
    for rh in range(2):
        for rw in range(2):
            taps = [wb[:, :, 1 - 2 * sh + rh, 1 - 2 * sw + rw]
                    for sh in (rh - 1, rh) for sw in (rw - 1, rw)]
            phases.append(jnp.concatenate(taps, axis=0))
    return jnp.stack(phases, axis=0)


def _stats_to_ss(st, count, gamma, beta, eps=1e-5):
    s = jnp.sum(st[:, 0, :], axis=0)
    q = jnp.sum(st[:, 1, :], axis=0)
    mean = s / count
    var = q / count - mean * mean
    inv = jax.lax.rsqrt(var + eps)
    scale = gamma * inv
    return jnp.stack([scale, beta - mean * scale], axis=0).astype(jnp.float32)


def kernel(latent, attr, w1, w2, w3, w4, w5, g1, b1, g2, b2, g3, b3, g4, b4):
    B = latent.shape[0]
    x = jnp.concatenate([latent, attr], axis=1).astype(jnp.float32)

    wm1 = _w_to_mat(w1)
    wph = {i: _w_to_phase_mats(w) for i, w in ((2, w2), (3, w3), (4, w4))}
    cin5 = w5.shape[0]
    wt5 = jnp.transpose(w5.astype(jnp.bfloat16),
                        (2, 3, 1, 0)).reshape(48, cin5)
    z48 = jnp.zeros((48, cin5), jnp.bfloat16)
    wt5 = jnp.concatenate([jnp.concatenate([wt5, z48], axis=1),
                           jnp.concatenate([z48, wt5], axis=1)], axis=0)
    emat = _phase_interleave_matrix()

    C1 = w1.shape[1]
    y, st = _dense_bn_layer(x.astype(jnp.bfloat16), wm1, C1)
    ss = _stats_to_ss(st, B * 16, g1, b1)

    H, W, Cin = 4, 4, C1
    for wmat, g, b in ((wph[2], g2, b2), (wph[3], g3, b3), (wph[4], g4, b4)):
        y, st = _convt_bn_layer(y, wmat, ss, B, H, W, Cin)
        H, W, Cin = 2 * H, 2 * W, Cin // 2
        ss = _stats_to_ss(st, B * H * W, g, b)

    y5 = _rgb_layer(y, wt5, ss, emat, B, H, W, Cin)
    return jnp.transpose(y5, (1, 0, 2, 3))


# speedup vs baseline: 5.0457x; 1.0045x over previous
import functools

import jax
import jax.numpy as jnp
import numpy as np
from jax.experimental import pallas as pl
from jax.experimental.pallas import tpu as pltpu


def _dense_bn_kernel(x_ref, w_ref, o_ref, st_ref, *, C1):
    y = jnp.dot(x_ref[...], w_ref[...], preferred_element_type=jnp.float32)
    o_ref[...] = y
    acc_s = None
    acc_q = None
    for j in range(16):
        blk = y[:, j * C1:(j + 1) * C1]
        acc_s = blk if acc_s is None else acc_s + blk
        acc_q = blk * blk if acc_q is None else acc_q + blk * blk
    st_ref[0, 0, :] = jnp.sum(acc_s, axis=0)
    st_ref[0, 1, :] = jnp.sum(acc_q, axis=0)


def _convt_kernel(x_ref, w_ref, ss_ref, o_ref, st_ref, xsh_ref, *, H, W,
                  Cin, Cout, BB):
    M = BB * H * W

    x = x_ref[...]
    reps = x.shape[-1] // Cin
    ss = jnp.tile(ss_ref[...], (1, reps)) if reps > 1 else ss_ref[...]
    xn = jnp.maximum(x * ss[0, :] + ss[1, :], 0.0)
    xn = xn.reshape(BB, H, W, Cin)

    zcol = jnp.zeros((BB, H, 1, Cin), jnp.float32)
    xl = jnp.concatenate([xn[:, :, 1:, :], zcol], axis=2)
    xr = jnp.concatenate([zcol, xn[:, :, :W - 1, :]], axis=2)
    zrow = jnp.zeros((BB, W, 3 * Cin), jnp.bfloat16)
    xsh_ref[:, 0, :, :] = zrow
    xsh_ref[:, H + 1, :, :] = zrow
    xsh_ref[:, 1:H + 1, :, 0:Cin] = xr.astype(jnp.bfloat16)
    xsh_ref[:, 1:H + 1, :, Cin:2 * Cin] = xn.astype(jnp.bfloat16)
    xsh_ref[:, 1:H + 1, :, 2 * Cin:] = xl.astype(jnp.bfloat16)

    ysum = None
    ysq = None
    for rh in range(2):
        for rw in range(2):
            p = 2 * rh + rw
            lo = xsh_ref[:, rh:rh + H, :, rw * Cin:(rw + 2) * Cin]
            hi = xsh_ref[:, rh + 1:rh + 1 + H, :, rw * Cin:(rw + 2) * Cin]
            acc = jnp.dot(lo.reshape(M, 2 * Cin), w_ref[p, :2 * Cin, :],
                          preferred_element_type=jnp.float32)
            acc = acc + jnp.dot(hi.reshape(M, 2 * Cin), w_ref[p, 2 * Cin:, :],
                                preferred_element_type=jnp.float32)
            s = jnp.sum(acc, axis=0)
            q = jnp.sum(acc * acc, axis=0)
            ysum = s if ysum is None else ysum + s
            ysq = q if ysq is None else ysq + q
            o_ref[:, rh, :, rw * Cout:(rw + 1) * Cout] = (
                acc.reshape(BB * H, W, Cout))
    st_ref[0, 0, :] = ysum
    st_ref[0, 1, :] = ysq


def _rgb_kernel(x_ref, wt_ref, ss_ref, e_ref, o_ref, *, H, W, BB):
    M = BB * H * W
    MP = M // 2

    x = x_ref[...]
    reps = x.shape[-1] // ss_ref.shape[-1]
    ss = jnp.tile(ss_ref[...], (1, reps)) if reps > 1 else ss_ref[...]
    xn = jnp.maximum(x * ss[0, :] + ss[1, :], 0.0)
    xnp = xn.astype(wt_ref.dtype).reshape(MP, wt_ref.shape[1])

    pt = jax.lax.dot_general(wt_ref[...], xnp, (((1,), (1,)), ((), ())),
                             preferred_element_type=jnp.float32)

    lane = jax.lax.broadcasted_iota(jnp.int32, (1, MP), 1)
    j2 = lane & (W // 2 - 1)
    a_idx = (lane // (W // 2)) & (H - 1)

    rows = []
    for rh in range(2):
        for rw in range(2):
            accs = [None, None]
            for dh in range(2):
                for dw in range(2):
                    t = (3 - 2 * dh - rh) * 4 + (3 - 2 * dw - rw)
                    sh = dh + rh - 1
                    sw = dw + rw - 1
                    s = sh * W + sw
                    for q in range(2):
                        qs = (q + s) & 1
                        d = (q + s - qs) // 2
                        v = pt[48 * qs + 3 * t:48 * qs + 3 * t + 3, :]
                        if d:
                            v = pltpu.roll(v, (-d) % MP, axis=1)
                        ok = None
                        if sh:
                            ok = (a_idx + sh >= 0) & (a_idx + sh < H)
                        if sw:
                            jq = 2 * j2 + q
                            okw = (jq + sw >= 0) & (jq + sw < W)
                            ok = okw if ok is None else ok & okw
                        if ok is not None:
                            v = jnp.where(ok, v, 0.0)
                        accs[q] = v if accs[q] is None else accs[q] + v
            rows.append(jnp.tanh(accs[0]))
            rows.append(jnp.tanh(accs[1]))

    cat = jnp.concatenate(
        [r.astype(jnp.bfloat16).reshape(3 * MP // 128, 128) for r in rows],
        axis=-1)
    out = jnp.dot(cat, e_ref[...], preferred_element_type=jnp.float32)
    o_ref[...] = out.reshape(3, MP // 128, 1024)


def _dense_bn_layer(x, wmat, C1):
    B, K = x.shape
    N = wmat.shape[1]
    BB = B // 2
    out, st = pl.pallas_call(
        functools.partial(_dense_bn_kernel, C1=C1),
        out_shape=(jax.ShapeDtypeStruct((B, N), jnp.float32),
                   jax.ShapeDtypeStruct((2, 2, C1), jnp.float32)),
        grid=(2,),
        in_specs=[pl.BlockSpec((BB, K), lambda b: (b, 0)),
                  pl.BlockSpec((K, N), lambda b: (0, 0))],
        out_specs=(pl.BlockSpec((BB, N), lambda b: (b, 0)),
                   pl.BlockSpec((1, 2, C1), lambda b: (b, 0, 0))),
        compiler_params=pltpu.CompilerParams(dimension_semantics=("parallel",)),
    )(x, wmat)
    return out, st


def _pick_bb(B, H, W, Cin, Cout, cap_bytes=22 * 2 ** 20):
    bb = B // 2
    while bb > 1:
        m = bb * H * W
        out_b = bb * H * 2 * W * max(2 * Cout, 128) * 4 * 2
        in_b = m * max(Cin, 128) * 4 * 2
        scr_b = bb * (H + 2) * W * 3 * max(Cin, 128) * 2
        acc_b = m * max(Cout, 128) * 4 * 2
        if out_b + in_b + scr_b + acc_b <= cap_bytes:
            break
        bb //= 2
    return bb


def _convt_bn_layer(x_raw, wph, scale_shift, B, H, W, Cin):
    Cout = wph.shape[2]
    BB = _pick_bb(B, H, W, Cin, Cout)
    G = B // BB

    xdims = x_raw.shape
    xblock = (xdims[0] // G,) + xdims[1:]
    nlead = len(xdims) - 1

    res = pl.pallas_call(
        functools.partial(_convt_kernel, H=H, W=W, Cin=Cin, Cout=Cout, BB=BB),
        out_shape=(jax.ShapeDtypeStruct((B * H, 2, W, 2 * Cout), jnp.float32),
                   jax.ShapeDtypeStruct((G, 2, Cout), jnp.float32)),
        grid=(G,),
        in_specs=[
            pl.BlockSpec(xblock, lambda b: (b,) + (0,) * nlead),
            pl.BlockSpec((4, 4 * Cin, Cout), lambda b: (0, 0, 0)),
            pl.BlockSpec((2, Cin), lambda b: (0, 0)),
        ],
        out_specs=(pl.BlockSpec((BB * H, 2, W, 2 * Cout),
                                lambda b: (b, 0, 0, 0)),
                   pl.BlockSpec((1, 2, Cout), lambda b: (b, 0, 0))),
        scratch_shapes=[pltpu.VMEM((BB, H + 2, W, 3 * Cin), jnp.bfloat16)],
        compiler_params=pltpu.CompilerParams(dimension_semantics=("parallel",)),
    )(x_raw, wph, scale_shift)

    return res[0], res[1]


def _phase_interleave_matrix():
    e = np.zeros((1024, 1024), np.float32)
    for q_out in range(1024):
        da, rem = divmod(q_out, 128)
        rh, u = divmod(rem, 64)
        j, rw = divmod(u, 2)
        p = 2 * rh + rw
        v = 32 * da + j
        e[128 * (2 * p + (v & 1)) + v // 2, q_out] = 1.0
    return jnp.asarray(e).astype(jnp.bfloat16)


def _rgb_layer(x_raw, wt, scale_shift, emat, B, H, W, Cin):
    BB = 8 if B % 8 == 0 else (4 if B % 4 == 0 else B // 2)
    G = B // BB
    M = BB * H * W

    xdims = x_raw.shape
    xblock = (xdims[0] // G,) + xdims[1:]
    nlead = len(xdims) - 1

    out = pl.pallas_call(
        functools.partial(_rgb_kernel, H=H, W=W, BB=BB),
        out_shape=jax.ShapeDtypeStruct((3, B * (H * W) // 256, 1024),
                                       jnp.float32),
        grid=(G,),
        in_specs=[
            pl.BlockSpec(xblock, lambda b: (b,) + (0,) * nlead),
            pl.BlockSpec(wt.shape, lambda b: (0, 0)),
            pl.BlockSpec((2, Cin), lambda b: (0, 0)),
            pl.BlockSpec((1024, 1024), lambda b: (0, 0)),
        ],
        out_specs=pl.BlockSpec((3, M // 256, 1024), lambda b: (0, b, 0)),
        compiler_params=pltpu.CompilerParams(dimension_semantics=("parallel",)),
    )(x_raw, wt, scale_shift, emat)

    return out.reshape(3, B, 2 * H, 2 * W)


def _w_to_mat(w):
    cin, cout = w.shape[0], w.shape[1]
    wb = w.astype(jnp.bfloat16)
    return jnp.transpose(wb, (0, 2, 3, 1)).reshape(cin, 16 * cout)


def _w_to_phase_mats(w):
    wb = w.astype(jnp.bfloat16)
    phases = []
    for rh in range(2):
        for rw in range(2):
            taps = [wb[:, :, 1 - 2 * sh + rh, 1 - 2 * sw + rw]
                    for sh in (rh - 1, rh) for sw in (rw - 1, rw)]
            phases.append(jnp.concatenate(taps, axis=0))
    return jnp.stack(phases, axis=0)


def _stats_to_ss(st, count, gamma, beta, eps=1e-5):
    s = jnp.sum(st[:, 0, :], axis=0)
    q = jnp.sum(st[:, 1, :], axis=0)
    mean = s / count
    var = q / count - mean * mean
    inv = jax.lax.rsqrt(var + eps)
    scale = gamma * inv
    return jnp.stack([scale, beta - mean * scale], axis=0).astype(jnp.float32)


def kernel(latent, attr, w1, w2, w3, w4, w5, g1, b1, g2, b2, g3, b3, g4, b4):
    B = latent.shape[0]
    x = jnp.concatenate([latent, attr], axis=1).astype(jnp.float32)

    wm1 = _w_to_mat(w1)
    wph = {i: _w_to_phase_mats(w) for i, w in ((2, w2), (3, w3), (4, w4))}
    cin5 = w5.shape[0]
    wt5 = jnp.transpose(w5.astype(jnp.bfloat16),
                        (2, 3, 1, 0)).reshape(48, cin5)
    z48 = jnp.zeros((48, cin5), jnp.bfloat16)
    wt5 = jnp.concatenate([jnp.concatenate([wt5, z48], axis=1),
                           jnp.concatenate([z48, wt5], axis=1)], axis=0)
    emat = _phase_interleave_matrix()

    C1 = w1.shape[1]
    y, st = _dense_bn_layer(x.astype(jnp.bfloat16), wm1, C1)
    ss = _stats_to_ss(st, B * 16, g1, b1)

    H, W, Cin = 4, 4, C1
    for wmat, g, b in ((wph[2], g2, b2), (wph[3], g3, b3), (wph[4], g4, b4)):
        y, st = _convt_bn_layer(y, wmat, ss, B, H, W, Cin)
        H, W, Cin = 2 * H, 2 * W, Cin // 2
        ss = _stats_to_ss(st, B * H * W, g, b)

    y5 = _rgb_layer(y, wt5, ss, emat, B, H, W, Cin)
    return jnp.transpose(y5, (1, 0, 2, 3))
